# Initial kernel scaffold; baseline (speedup 1.0000x reference)
import jax
import jax.numpy as jnp
from jax.experimental import pallas as pl


def kernel(user_feat, item_feat, edge_src_item, edge_dst_user, edge_order_by, edge_src_user, edge_dst_item, edge_order_pby, last_item_idx, last_user_idx, W_user, W_item, agg_gate_u, agg_gate_i, last_weight_u, last_weight_i, u_time_k, u_time_v, i_time_k, i_time_v):
    raise NotImplementedError("write your pallas kernel here")


# speedup vs baseline: 2.0016x; 2.0016x over previous
import functools
import jax
import jax.numpy as jnp
import numpy as np
from jax import lax
from jax.experimental import pallas as pl
from jax.experimental.pallas import tpu as pltpu
from jax.experimental.pallas import tpu_sc as plsc

N = 10000
E = 320000
D = 128
T = 50
NC = 2
NS = 16
NW = NC * NS
CH = E // NW
B = 80
NB = CH // B
WID = D + 8
SCALE = 1.0 / np.sqrt(D)

_mesh = plsc.VectorSubcoreMesh(core_axis_name="c", subcore_axis_name="s")


@functools.partial(
    pl.kernel,
    out_type=jax.ShapeDtypeStruct((NC, N, WID), jnp.float32),
    mesh=_mesh,
    scratch_types=[
        pltpu.VMEM_SHARED((N, WID), jnp.float32),
        pltpu.VMEM((B, D), jnp.float32),
        pltpu.VMEM((B, D), jnp.float32),
        pltpu.VMEM((B, WID), jnp.float32),
        pltpu.VMEM((B,), jnp.int32),
        pltpu.VMEM((B,), jnp.int32),
        pltpu.VMEM((B,), jnp.int32),
        pltpu.VMEM((T, D), jnp.float32),
        pltpu.VMEM((T, D), jnp.float32),
        pltpu.SemaphoreType.DMA,
        pltpu.SemaphoreType.DMA,
    ],
    compiler_params=pltpu.CompilerParams(needs_layout_passes=False,
                                         use_tc_tiling_on_sc=False),
)
def _sc_relation(q_hbm, k_hbm, tk_hbm, tv_hbm, src_hbm, dst_hbm, ord_hbm,
                 out_hbm, w_sh, kbuf, qbuf, wv, src_v, dst_v, ord_v,
                 tkv, tvv, sem1, sem2):
    c = lax.axis_index("c")
    s = lax.axis_index("s")
    wid = c * NS + s
    iota16 = lax.iota(jnp.int32, 16)
    zero16 = jnp.zeros((16,), jnp.float32)

    pltpu.sync_copy(tk_hbm, tkv)
    pltpu.sync_copy(tv_hbm, tvv)

    def _zb(i, tok):
        lin = jnp.full((16,), i * 16, jnp.int32) + iota16
        plsc.store_scatter(wv, [lin // WID, lin % WID], zero16)
        return tok
    lax.fori_loop(0, B * WID // 16, _zb, 0)

    zbase = s * 624

    @pl.when(s < NS - 1)
    def _zero_main():
        for j in range(7):
            pltpu.sync_copy(wv, w_sh.at[pl.ds(zbase + j * B, B)])
        pltpu.sync_copy(wv.at[pl.ds(0, 64)], w_sh.at[pl.ds(zbase + 560, 64)])

    @pl.when(s == NS - 1)
    def _zero_last():
        for j in range(8):
            pltpu.sync_copy(wv, w_sh.at[pl.ds(zbase + j * B, B)])

    plsc.subcore_barrier()

    def _block(b, tok):
        base = wid * CH + b * B
        pltpu.sync_copy(src_hbm.at[pl.ds(base, B)], src_v)
        pltpu.sync_copy(dst_hbm.at[pl.ds(base, B)], dst_v)
        pltpu.sync_copy(ord_hbm.at[pl.ds(base, B)], ord_v)
        cp1 = pltpu.async_copy(k_hbm.at[src_v], kbuf, sem1)
        cp2 = pltpu.async_copy(q_hbm.at[dst_v], qbuf, sem2)
        cp1.wait()
        cp2.wait()
        for g in range(B // 16):
            rows = g * 16 + iota16
            ordv = ord_v[pl.ds(g * 16, 16)]

            def _dot(dd, acc):
                cols = jnp.full((16,), dd, jnp.int32)
                kv = plsc.load_gather(kbuf, [rows, cols])
                qv = plsc.load_gather(qbuf, [rows, cols])
                tkx = plsc.load_gather(tkv, [ordv, cols])
                return acc + qv * (kv + tkx)

            e = lax.fori_loop(0, D, _dot, zero16) * SCALE
            ex = jnp.exp(e)
            plsc.store_scatter(wv, [rows, jnp.full((16,), D, jnp.int32)], ex)

            def _wval(dd, t):
                cols = jnp.full((16,), dd, jnp.int32)
                kv = plsc.load_gather(kbuf, [rows, cols])
                tvx = plsc.load_gather(tvv, [ordv, cols])
                plsc.store_scatter(wv, [rows, cols], ex * (kv + tvx))
                return t

            lax.fori_loop(0, D, _wval, 0)
        pltpu.sync_copy(wv, w_sh.at[dst_v], add=True)
        return tok

    lax.fori_loop(0, NB, _block, 0)
    plsc.subcore_barrier()

    @pl.when(s < NS - 1)
    def _out_main():
        pltpu.sync_copy(w_sh.at[pl.ds(zbase, 624)],
                        out_hbm.at[c, pl.ds(zbase, 624)])

    @pl.when(s == NS - 1)
    def _out_last():
        pltpu.sync_copy(w_sh.at[pl.ds(zbase, 640)],
                        out_hbm.at[c, pl.ds(zbase, 640)])


_GB = 104
_GROWS = 312
@functools.partial(
    pl.kernel,
    out_type=jax.ShapeDtypeStruct((N, D), jnp.float32),
    mesh=_mesh,
    scratch_types=[
        pltpu.VMEM((_GB,), jnp.int32),
        pltpu.VMEM((_GB, D), jnp.float32),
        pltpu.VMEM((16,), jnp.int32),
        pltpu.VMEM((16, D), jnp.float32),
        pltpu.SemaphoreType.DMA,
    ],
    compiler_params=pltpu.CompilerParams(needs_layout_passes=False),
)
def _sc_gather(tab_hbm, idx_hbm, out_hbm, idx_v, rows_v, idx16, rows16, sem):
    c = lax.axis_index("c")
    s = lax.axis_index("s")
    wid = c * NS + s
    for j in range(_GROWS // _GB):
        base = wid * _GROWS + j * _GB
        pltpu.sync_copy(idx_hbm.at[pl.ds(base, _GB)], idx_v)
        pltpu.async_copy(tab_hbm.at[idx_v], rows_v, sem).wait()
        pltpu.sync_copy(rows_v, out_hbm.at[pl.ds(base, _GB)])

    @pl.when(wid == 0)
    def _tail():
        pltpu.sync_copy(idx_hbm.at[pl.ds(NW * _GROWS, 16)], idx16)
        pltpu.async_copy(tab_hbm.at[idx16], rows16, sem).wait()
        pltpu.sync_copy(rows16, out_hbm.at[pl.ds(NW * _GROWS, 16)])


_RB = 1000


def _tc_matmul_body(x_ref, w_ref, o_ref):
    o_ref[...] = jnp.dot(x_ref[...], w_ref[...],
                         preferred_element_type=jnp.float32)


def _tc_matmul(x, w):
    return pl.pallas_call(
        _tc_matmul_body,
        grid=(N // _RB,),
        in_specs=[
            pl.BlockSpec((_RB, D), lambda i: (i, 0)),
            pl.BlockSpec((D, D), lambda i: (0, 0)),
        ],
        out_specs=pl.BlockSpec((_RB, D), lambda i: (i, 0)),
        out_shape=jax.ShapeDtypeStruct((N, D), jnp.float32),
    )(x, w)


def _elu(x):
    return jnp.where(x > 0, x, jnp.exp(x) - 1.0)


def _tc_final_body(wacc_ref, rows_ref, feat_ref, lw_ref, g_ref, o_ref):
    w = wacc_ref[0] + wacc_ref[1]
    den = w[:, D:D + 1]
    longv = w[:, :D] / (den + 1e-9)
    short = _elu(jnp.dot(rows_ref[...], lw_ref[...],
                         preferred_element_type=jnp.float32))
    new = (jnp.dot(longv, g_ref[:D], preferred_element_type=jnp.float32)
           + jnp.dot(short, g_ref[D:], preferred_element_type=jnp.float32))
    o_ref[...] = _elu(new + feat_ref[...])


def _tc_final(wacc, rows, feat, lw, gate):
    return pl.pallas_call(
        _tc_final_body,
        grid=(N // _RB,),
        in_specs=[
            pl.BlockSpec((NC, _RB, WID), lambda i: (0, i, 0)),
            pl.BlockSpec((_RB, D), lambda i: (i, 0)),
            pl.BlockSpec((_RB, D), lambda i: (i, 0)),
            pl.BlockSpec((D, D), lambda i: (0, 0)),
            pl.BlockSpec((2 * D, D), lambda i: (0, 0)),
        ],
        out_specs=pl.BlockSpec((_RB, D), lambda i: (i, 0)),
        out_shape=jax.ShapeDtypeStruct((N, D), jnp.float32),
    )(wacc, rows, feat, lw, gate)


def kernel(user_feat, item_feat, edge_src_item, edge_dst_user, edge_order_by,
           edge_src_user, edge_dst_item, edge_order_pby, last_item_idx,
           last_user_idx, W_user, W_item, agg_gate_u, agg_gate_i,
           last_weight_u, last_weight_i, u_time_k, u_time_v, i_time_k,
           i_time_v):
    user_h = _tc_matmul(user_feat, W_user)
    item_h = _tc_matmul(item_feat, W_item)

    i32 = lambda x: x.astype(jnp.int32)
    wacc_u = _sc_relation(user_h, item_h, u_time_k, u_time_v,
                          i32(edge_src_item), i32(edge_dst_user),
                          i32(edge_order_by))
    wacc_i = _sc_relation(item_h, user_h, i_time_k, i_time_v,
                          i32(edge_src_user), i32(edge_dst_item),
                          i32(edge_order_pby))
    rows_u = _sc_gather(item_h, i32(last_item_idx))
    rows_i = _sc_gather(user_h, i32(last_user_idx))

    user_out = _tc_final(wacc_u, rows_u, user_feat, last_weight_u, agg_gate_u)
    item_out = _tc_final(wacc_i, rows_i, item_feat, last_weight_i, agg_gate_i)
    return (user_out, item_out)
